# baseline (speedup 1.0000x reference)
import functools

import jax
import jax.numpy as jnp
from jax import lax
from jax.experimental import pallas as pl
from jax.experimental.pallas import tpu as pltpu
from jax.experimental.pallas import tpu_sc as plsc

N = 10000
E = 320000
D = 128
DH = 64
NS = 16
EPT = E // NS
K = 128
NCH = -(-EPT // K)
PADE = NCH * K
RCH = 40
NRC = N // RCH
LR_SLOPE = 0.2


def _project(x, W, a2):
    R = 1000

    def body(x_ref, w_ref, a_ref, h2_ref, sd_ref):
        h = jnp.dot(x_ref[...], w_ref[...], preferred_element_type=jnp.float32)
        h2_ref[0] = h[:, :DH]
        h2_ref[1] = h[:, DH:]
        sd_ref[...] = jnp.dot(h, a_ref[...], preferred_element_type=jnp.float32)

    return pl.pallas_call(
        body,
        grid=(N // R,),
        in_specs=[
            pl.BlockSpec((R, D), lambda i: (i, 0)),
            pl.BlockSpec((D, D), lambda i: (0, 0)),
            pl.BlockSpec((D, 2), lambda i: (0, 0)),
        ],
        out_specs=[
            pl.BlockSpec((2, R, DH), lambda i: (0, i, 0)),
            pl.BlockSpec((R, 2), lambda i: (i, 0)),
        ],
        out_shape=[
            jax.ShapeDtypeStruct((2, N, DH), jnp.float32),
            jax.ShapeDtypeStruct((N, 2), jnp.float32),
        ],
    )(x, W, a2)


def _sc_gat(hflat, s, d, srcp, dstp, b):
    mesh = plsc.VectorSubcoreMesh(
        core_axis_name="c", subcore_axis_name="s", num_cores=2, num_subcores=NS
    )

    @functools.partial(
        pl.kernel,
        mesh=mesh,
        compiler_params=pltpu.CompilerParams(use_tc_tiling_on_sc=False, needs_layout_passes=False),
        out_type=jax.ShapeDtypeStruct((N, D), jnp.float32),
        scratch_types=[
            pltpu.VMEM((N,), jnp.float32),
            pltpu.VMEM((N,), jnp.float32),
            pltpu.VMEM((PADE,), jnp.int32),
            pltpu.VMEM((PADE,), jnp.int32),
            [pltpu.VMEM((K,), jnp.int32) for _ in range(3)],
            [pltpu.VMEM((K,), jnp.int32) for _ in range(3)],
            [pltpu.VMEM((K,), jnp.float32) for _ in range(3)],
            [pltpu.VMEM((K, DH), jnp.float32) for _ in range(3)],
            pltpu.VMEM((D,), jnp.float32),
            pltpu.VMEM((RCH, DH), jnp.float32),
            pltpu.VMEM((RCH,), jnp.float32),
            pltpu.VMEM_SHARED((N, DH), jnp.float32),
            pltpu.VMEM_SHARED((N,), jnp.float32),
            [pltpu.SemaphoreType.DMA for _ in range(3)],
            [pltpu.SemaphoreType.DMA for _ in range(3)],
        ],
    )
    def k(h_hbm, s_hbm, d_hbm, srcp_hbm, dstp_hbm, b_hbm, out_hbm,
          s_v, d_v, srcall, dstall, dstb, gidx, w_v, rows, bias_v,
          fin_h, fin_d, acc_h, acc_d, sem, sems):
        cid = lax.axis_index("c")
        sid = lax.axis_index("s")

        pltpu.sync_copy(s_hbm, s_v)
        pltpu.sync_copy(d_hbm, d_v)
        pltpu.sync_copy(b_hbm, bias_v)
        pltpu.sync_copy(srcp_hbm.at[sid], srcall.at[pl.ds(0, EPT)])
        pltpu.sync_copy(dstp_hbm.at[sid], dstall.at[pl.ds(0, EPT)])

        def smax_body(i, m):
            return jnp.maximum(m, jnp.max(s_v[pl.ds(i * 16, 16)]))

        S = lax.fori_loop(0, N // 16, smax_body, jnp.float32(-jnp.inf))

        zero16 = jnp.zeros((16,), jnp.float32)

        def zh_body(r, _):
            for c in range(DH // 16):
                fin_h[r, pl.ds(c * 16, 16)] = zero16
            return 0

        lax.fori_loop(0, RCH, zh_body, 0)

        fin_d[pl.ds(0, 16)] = zero16
        fin_d[pl.ds(16, 16)] = zero16
        fin_d[pl.ds(24, 16)] = zero16

        def zcopy_body(j, _):
            cidx = j * NS + sid

            @pl.when(cidx < NRC)
            def _():
                r0 = cidx * RCH
                pltpu.sync_copy(fin_h, acc_h.at[pl.ds(r0, RCH)])
                pltpu.sync_copy(fin_d, acc_d.at[pl.ds(r0, RCH)])

            return 0

        lax.fori_loop(0, -(-NRC // NS), zcopy_body, 0)
        plsc.subcore_barrier()

        coff = cid * N

        def prep(ch, p):
            base = ch * K

            @functools.partial(plsc.parallel_loop, 0, K // 16)
            def _w_body(i):
                sl = pl.ds(i * 16, 16)
                gsl = pl.ds(base + i * 16, 16)
                srcv = jnp.clip(srcall[gsl], 0, N - 1)
                dstv = jnp.clip(dstall[gsl], 0, N - 1)
                sv = plsc.load_gather(s_v, [srcv])
                dv = plsc.load_gather(d_v, [dstv])
                e = sv + dv
                elr = jnp.where(e > 0, e, LR_SLOPE * e)
                m = S + dv
                mlr = jnp.where(m > 0, m, LR_SLOPE * m)
                w = jnp.exp(elr - mlr)
                lane = base + i * 16 + lax.iota(jnp.int32, 16)
                w_v[p][sl] = jnp.where(lane < EPT, w, 0.0)
                gidx[p][sl] = srcv + coff
                dstb[p][sl] = dstv

        def gather_start(p):
            pltpu.async_copy(h_hbm.at[gidx[p]], rows[p], sem[p])

        def gather_wait(p):
            pltpu.make_async_copy(h_hbm.at[gidx[p]], rows[p], sem[p]).wait()

        def scatter_wait(p):
            pltpu.make_async_copy(rows[p], acc_h.at[dstb[p]], sems[p]).wait()
            pltpu.make_async_copy(w_v[p], acc_d.at[dstb[p]], sems[p]).wait()

        def process(p):
            @functools.partial(plsc.parallel_loop, 0, K // 16)
            def _scale(g):
                wv = w_v[p][pl.ds(g * 16, 16)]
                for l in range(16):
                    wk = wv[l]
                    kk = g * 16 + l
                    for c in range(DH // 16):
                        sl = pl.ds(c * 16, 16)
                        rows[p][kk, sl] = rows[p][kk, sl] * wk
            pltpu.async_copy(rows[p], acc_h.at[dstb[p]], sems[p], add=True)
            pltpu.async_copy(w_v[p], acc_d.at[dstb[p]], sems[p], add=True)

        prep(0, 0)
        gather_start(0)
        prep(1, 1)
        gather_start(1)

        def tri_body(t, _):
            for j in range(3):
                ch = t * 3 + j

                @pl.when(ch < NCH)
                def _():
                    gather_wait(j)
                    process(j)

                @pl.when(ch + 2 < NCH)
                def _():
                    pn = (j + 2) % 3

                    @pl.when(ch >= 1)
                    def _():
                        scatter_wait(pn)

                    prep(ch + 2, pn)
                    gather_start(pn)

            return 0

        lax.fori_loop(0, -(-NCH // 3), tri_body, 0)
        for p in range(3):
            scatter_wait(p)
        plsc.subcore_barrier()

        b0 = bias_v[pl.ds(cid * DH, 16)]
        b1 = bias_v[pl.ds(cid * DH + 16, 16)]
        b2 = bias_v[pl.ds(cid * DH + 32, 16)]
        b3 = bias_v[pl.ds(cid * DH + 48, 16)]

        def fin_body(j, _):
            cidx = j * NS + sid

            @pl.when(cidx < NRC)
            def _():
                r0 = cidx * RCH
                pltpu.sync_copy(acc_h.at[pl.ds(r0, RCH)], fin_h)
                pltpu.sync_copy(acc_d.at[pl.ds(r0, RCH)], fin_d)

                for base, lo in ((0, 0), (16, 0), (24, 8)):
                    invv = 1.0 / (fin_d[pl.ds(base, 16)] + 1e-16)
                    for l in range(lo, 16):
                        inv = invv[l]
                        r = base + l
                        fin_h[r, pl.ds(0, 16)] = fin_h[r, pl.ds(0, 16)] * inv + b0
                        fin_h[r, pl.ds(16, 16)] = fin_h[r, pl.ds(16, 16)] * inv + b1
                        fin_h[r, pl.ds(32, 16)] = fin_h[r, pl.ds(32, 16)] * inv + b2
                        fin_h[r, pl.ds(48, 16)] = fin_h[r, pl.ds(48, 16)] * inv + b3

                pltpu.sync_copy(fin_h, out_hbm.at[pl.ds(r0, RCH), pl.ds(cid * DH, DH)])

            return 0

        lax.fori_loop(0, -(-NRC // NS), fin_body, 0)

    return k(hflat, s, d, srcp, dstp, b)


def kernel(x, edge_index, W, a_src, a_dst, b):
    src = edge_index[0].astype(jnp.int32)
    dst = edge_index[1].astype(jnp.int32)
    a2 = jnp.stack([a_src, a_dst], axis=1)
    h2, sd = _project(x, W, a2)
    hflat = h2.reshape(2 * N, DH)
    return _sc_gat(hflat, sd[:, 0], sd[:, 1], src.reshape(NS, EPT), dst.reshape(NS, EPT), b)


# speedup vs baseline: 63.0161x; 63.0161x over previous
import functools

import jax
import jax.numpy as jnp
from jax import lax
from jax.experimental import pallas as pl
from jax.experimental.pallas import tpu as pltpu
from jax.experimental.pallas import tpu_sc as plsc

N = 10000
E = 320000
D = 128
DH = 64
NS = 16
EPT = E // NS
K = 128
NCH = -(-EPT // K)
PADE = NCH * K
RCH = 40
NRC = N // RCH
LR_SLOPE = 0.2


def _project(x, W, a2):
    R = 1000

    def body(x_ref, w_ref, a_ref, h2_ref, sd_ref):
        h = jnp.dot(x_ref[...], w_ref[...], preferred_element_type=jnp.float32)
        h2_ref[0] = h[:, :DH]
        h2_ref[1] = h[:, DH:]
        sd_ref[...] = jnp.dot(h, a_ref[...], preferred_element_type=jnp.float32)

    return pl.pallas_call(
        body,
        grid=(N // R,),
        in_specs=[
            pl.BlockSpec((R, D), lambda i: (i, 0)),
            pl.BlockSpec((D, D), lambda i: (0, 0)),
            pl.BlockSpec((D, 2), lambda i: (0, 0)),
        ],
        out_specs=[
            pl.BlockSpec((2, R, DH), lambda i: (0, i, 0)),
            pl.BlockSpec((R, 2), lambda i: (i, 0)),
        ],
        out_shape=[
            jax.ShapeDtypeStruct((2, N, DH), jnp.float32),
            jax.ShapeDtypeStruct((N, 2), jnp.float32),
        ],
    )(x, W, a2)


def _sc_gat(hflat, s, d, srcp, dstp, b):
    mesh = plsc.VectorSubcoreMesh(
        core_axis_name="c", subcore_axis_name="s", num_cores=2, num_subcores=NS
    )

    @functools.partial(
        pl.kernel,
        mesh=mesh,
        compiler_params=pltpu.CompilerParams(use_tc_tiling_on_sc=False, needs_layout_passes=False),
        out_type=jax.ShapeDtypeStruct((N, D), jnp.float32),
        scratch_types=[
            pltpu.VMEM((N,), jnp.float32),
            pltpu.VMEM((N,), jnp.float32),
            pltpu.VMEM((PADE,), jnp.int32),
            pltpu.VMEM((PADE,), jnp.int32),
            [pltpu.VMEM((K,), jnp.int32) for _ in range(3)],
            [pltpu.VMEM((K,), jnp.int32) for _ in range(3)],
            [pltpu.VMEM((K,), jnp.float32) for _ in range(3)],
            [pltpu.VMEM((K, DH), jnp.float32) for _ in range(3)],
            pltpu.VMEM((D,), jnp.float32),
            pltpu.VMEM((RCH, DH), jnp.float32),
            pltpu.VMEM((RCH,), jnp.float32),
            pltpu.VMEM_SHARED((N, DH), jnp.float32),
            pltpu.VMEM_SHARED((N,), jnp.float32),
            [pltpu.SemaphoreType.DMA for _ in range(3)],
            [pltpu.SemaphoreType.DMA for _ in range(3)],
        ],
    )
    def k(h_hbm, s_hbm, d_hbm, srcp_hbm, dstp_hbm, b_hbm, out_hbm,
          s_v, d_v, srcall, dstall, dstb, gidx, w_v, rows, bias_v,
          fin_h, fin_d, acc_h, acc_d, sem, sems):
        cid = lax.axis_index("c")
        sid = lax.axis_index("s")

        pltpu.sync_copy(s_hbm, s_v)
        pltpu.sync_copy(d_hbm, d_v)
        pltpu.sync_copy(b_hbm, bias_v)
        pltpu.sync_copy(srcp_hbm.at[sid], srcall.at[pl.ds(0, EPT)])
        pltpu.sync_copy(dstp_hbm.at[sid], dstall.at[pl.ds(0, EPT)])

        def smax_body(i, m):
            return jnp.maximum(m, jnp.max(s_v[pl.ds(i * 16, 16)]))

        S = lax.fori_loop(0, N // 16, smax_body, jnp.float32(-jnp.inf))

        zero16 = jnp.zeros((16,), jnp.float32)

        def zh_body(r, _):
            for c in range(DH // 16):
                fin_h[r, pl.ds(c * 16, 16)] = zero16
            return 0

        lax.fori_loop(0, RCH, zh_body, 0)

        fin_d[pl.ds(0, 16)] = zero16
        fin_d[pl.ds(16, 16)] = zero16
        fin_d[pl.ds(24, 16)] = zero16

        def zcopy_body(j, _):
            cidx = j * NS + sid

            @pl.when(cidx < NRC)
            def _():
                r0 = cidx * RCH
                pltpu.sync_copy(fin_h, acc_h.at[pl.ds(r0, RCH)])
                pltpu.sync_copy(fin_d, acc_d.at[pl.ds(r0, RCH)])

            return 0

        lax.fori_loop(0, -(-NRC // NS), zcopy_body, 0)
        plsc.subcore_barrier()

        coff = cid * N

        def prep(ch, p):
            base = ch * K

            def w_body(i, _):
                sl = pl.ds(i * 16, 16)
                gsl = pl.ds(base + i * 16, 16)
                srcv = jnp.clip(srcall[gsl], 0, N - 1)
                dstv = jnp.clip(dstall[gsl], 0, N - 1)
                sv = plsc.load_gather(s_v, [srcv])
                dv = plsc.load_gather(d_v, [dstv])
                e = sv + dv
                elr = jnp.where(e > 0, e, LR_SLOPE * e)
                m = S + dv
                mlr = jnp.where(m > 0, m, LR_SLOPE * m)
                w = jnp.exp(elr - mlr)
                lane = base + i * 16 + lax.iota(jnp.int32, 16)
                w_v[p][sl] = jnp.where(lane < EPT, w, 0.0)
                gidx[p][sl] = srcv + coff
                dstb[p][sl] = dstv
                return 0

            lax.fori_loop(0, K // 16, w_body, 0, unroll=2)

        def gather_start(p):
            pltpu.async_copy(h_hbm.at[gidx[p]], rows[p], sem[p])

        def gather_wait(p):
            pltpu.make_async_copy(h_hbm.at[gidx[p]], rows[p], sem[p]).wait()

        def scatter_wait(p):
            pltpu.make_async_copy(rows[p], acc_h.at[dstb[p]], sems[p]).wait()
            pltpu.make_async_copy(w_v[p], acc_d.at[dstb[p]], sems[p]).wait()

        def process(p):
            @functools.partial(plsc.parallel_loop, 0, K // 16)
            def _scale(g):
                wv = w_v[p][pl.ds(g * 16, 16)]
                for l in range(16):
                    wk = wv[l]
                    kk = g * 16 + l
                    for c in range(DH // 16):
                        sl = pl.ds(c * 16, 16)
                        rows[p][kk, sl] = rows[p][kk, sl] * wk
            pltpu.async_copy(rows[p], acc_h.at[dstb[p]], sems[p], add=True)
            pltpu.async_copy(w_v[p], acc_d.at[dstb[p]], sems[p], add=True)

        prep(0, 0)
        gather_start(0)
        prep(1, 1)
        gather_start(1)

        def tri_body(t, _):
            for j in range(3):
                ch = t * 3 + j

                @pl.when(ch < NCH)
                def _():
                    gather_wait(j)
                    process(j)

                @pl.when(ch + 2 < NCH)
                def _():
                    pn = (j + 2) % 3

                    @pl.when(ch >= 1)
                    def _():
                        scatter_wait(pn)

                    prep(ch + 2, pn)
                    gather_start(pn)

            return 0

        lax.fori_loop(0, -(-NCH // 3), tri_body, 0)
        for p in range(3):
            scatter_wait(p)
        plsc.subcore_barrier()

        b0 = bias_v[pl.ds(cid * DH, 16)]
        b1 = bias_v[pl.ds(cid * DH + 16, 16)]
        b2 = bias_v[pl.ds(cid * DH + 32, 16)]
        b3 = bias_v[pl.ds(cid * DH + 48, 16)]

        def fin_body(j, _):
            cidx = j * NS + sid

            @pl.when(cidx < NRC)
            def _():
                r0 = cidx * RCH
                pltpu.sync_copy(acc_h.at[pl.ds(r0, RCH)], fin_h)
                pltpu.sync_copy(acc_d.at[pl.ds(r0, RCH)], fin_d)

                for base, lo in ((0, 0), (16, 0), (24, 8)):
                    invv = 1.0 / (fin_d[pl.ds(base, 16)] + 1e-16)
                    for l in range(lo, 16):
                        inv = invv[l]
                        r = base + l
                        fin_h[r, pl.ds(0, 16)] = fin_h[r, pl.ds(0, 16)] * inv + b0
                        fin_h[r, pl.ds(16, 16)] = fin_h[r, pl.ds(16, 16)] * inv + b1
                        fin_h[r, pl.ds(32, 16)] = fin_h[r, pl.ds(32, 16)] * inv + b2
                        fin_h[r, pl.ds(48, 16)] = fin_h[r, pl.ds(48, 16)] * inv + b3

                pltpu.sync_copy(fin_h, out_hbm.at[pl.ds(r0, RCH), pl.ds(cid * DH, DH)])

            return 0

        lax.fori_loop(0, -(-NRC // NS), fin_body, 0)

    return k(hflat, s, d, srcp, dstp, b)


def kernel(x, edge_index, W, a_src, a_dst, b):
    src = edge_index[0].astype(jnp.int32)
    dst = edge_index[1].astype(jnp.int32)
    a2 = jnp.stack([a_src, a_dst], axis=1)
    h2, sd = _project(x, W, a2)
    hflat = h2.reshape(2 * N, DH)
    return _sc_gat(hflat, sd[:, 0], sd[:, 1], src.reshape(NS, EPT), dst.reshape(NS, EPT), b)
